# baseline (speedup 1.0000x reference)
import functools

import jax
import jax.numpy as jnp
from jax import lax
from jax.experimental import pallas as pl
from jax.experimental.pallas import tpu as pltpu
from jax.experimental.pallas import tpu_sc as plsc

B = 256
L = 1024
PROJ_DIM = 64
NUM_AA = 20

_TOKENS = B * L

_info = plsc.get_sparse_core_info()
_NC = _info.num_cores
_NS = _info.num_subcores
_NW = _NC * _NS
_TOK_PER_W = _TOKENS // _NW

_CHUNK = 128
_NCHUNK = _TOK_PER_W // _CHUNK


def _sc_kernel(idx_hbm, wt_hbm, b_hbm, out_hbm,
               wt_v, b_v, table_v, table_sh, idx_v, rows_v, sem):
    sid = lax.axis_index("s")
    wid = sid * _NC + lax.axis_index("c")

    @pl.when(sid == 0)
    def _build_table():
        pltpu.sync_copy(wt_hbm, wt_v)
        pltpu.sync_copy(b_hbm, b_v)

        def build(i, carry):
            r0 = i >> 2
            c0 = (i & 3) * 16
            table_v[r0, pl.ds(c0, 16)] = (
                wt_v[r0, pl.ds(c0, 16)] + b_v[pl.ds(c0, 16)]
            )
            return carry

        lax.fori_loop(0, NUM_AA * 4, build, 0, unroll=False)
        pltpu.sync_copy(table_v, table_sh)

    plsc.subcore_barrier()

    base = wid * _TOK_PER_W

    def chunk(k, carry):
        off = base + k * _CHUNK
        pltpu.sync_copy(idx_hbm.at[pl.ds(off, _CHUNK)], idx_v)
        pltpu.async_copy(table_sh.at[idx_v], rows_v, sem).wait()
        pltpu.sync_copy(rows_v, out_hbm.at[pl.ds(off, _CHUNK)])
        return carry

    lax.fori_loop(0, _NCHUNK, chunk, 0, unroll=False)


@jax.jit
def kernel(indices, W, b):
    idx = indices.reshape(_TOKENS).astype(jnp.int32)
    w_t = W.T.reshape(NUM_AA, PROJ_DIM)
    mesh = plsc.VectorSubcoreMesh(core_axis_name="c", subcore_axis_name="s")
    out = pl.kernel(
        _sc_kernel,
        mesh=mesh,
        out_type=jax.ShapeDtypeStruct((_TOKENS, PROJ_DIM), jnp.float32),
        scratch_types=[
            pltpu.VMEM((NUM_AA, PROJ_DIM), jnp.float32),
            pltpu.VMEM((PROJ_DIM,), jnp.float32),
            pltpu.VMEM((NUM_AA, PROJ_DIM), jnp.float32),
            pltpu.VMEM_SHARED((NUM_AA, PROJ_DIM), jnp.float32),
            pltpu.VMEM((_CHUNK,), jnp.int32),
            pltpu.VMEM((_CHUNK, PROJ_DIM), jnp.float32),
            pltpu.SemaphoreType.DMA,
        ],
    )(idx, w_t, b)
    return out.reshape(B, L, PROJ_DIM)
